# speedup vs baseline: 1.4208x; 1.4208x over previous
import functools
import math

import jax
import jax.numpy as jnp
from jax import lax
from jax.experimental import pallas as pl
from jax.experimental.pallas import tpu as pltpu
from jax.experimental.pallas import tpu_sc as plsc

D_MODEL = 1024
SCALE = math.sqrt(D_MODEL)
L = 16
NC, NS = 2, 16
NW = NC * NS
B_TOTAL = 4 * 4096
BPW = B_TOTAL // NW
C = 32
NCHUNK = BPW // C
NBUF = 2

_mesh = plsc.VectorSubcoreMesh(core_axis_name="c", subcore_axis_name="s")


@functools.partial(
    pl.kernel,
    mesh=_mesh,
    out_type=jax.ShapeDtypeStruct((B_TOTAL, D_MODEL), jnp.float32),
    scratch_types=[
        pltpu.VMEM((BPW,), jnp.int32),
        pltpu.VMEM((NBUF, C, D_MODEL), jnp.float32),
        pltpu.SemaphoreType.DMA,
        pltpu.SemaphoreType.DMA,
    ],
)
def _emb_lookup(table_hbm, idx_hbm, out_hbm, idx_v, rows_v, gsem, osem):
    wid = lax.axis_index("s") * NC + lax.axis_index("c")
    base = wid * BPW

    pltpu.sync_copy(idx_hbm.at[pl.ds(base, BPW)], idx_v)

    def gather_start(g):
        return pltpu.async_copy(
            table_hbm.at[idx_v.at[pl.ds(g * C, C)]], rows_v.at[g % NBUF], gsem
        )

    def scale_buf(buf):
        def row_body(r, _):
            for j in range(D_MODEL // L):
                sl = pl.ds(j * L, L)
                rows_v[buf, r, sl] = rows_v[buf, r, sl] * SCALE
            return 0

        lax.fori_loop(0, C, row_body, 0)

    gathers = [None] * NCHUNK
    out_copies = [None] * NBUF
    gathers[0] = gather_start(0)
    for g in range(NCHUNK):
        buf = g % NBUF
        gathers[g].wait()
        if g + 1 < NCHUNK:
            nbuf = (g + 1) % NBUF
            if out_copies[nbuf] is not None:
                out_copies[nbuf].wait()
                out_copies[nbuf] = None
            gathers[g + 1] = gather_start(g + 1)
        scale_buf(buf)
        out_copies[buf] = pltpu.async_copy(
            rows_v.at[buf], out_hbm.at[pl.ds(base + g * C, C)], osem
        )
    for b in range(NBUF):
        if out_copies[b] is not None:
            out_copies[b].wait()


def kernel(x, table):
    idx = x.astype(jnp.int32).reshape(B_TOTAL)
    out = _emb_lookup(table, idx)
    return out.reshape(x.shape + (D_MODEL,))


# speedup vs baseline: 1.4844x; 1.0447x over previous
import functools
import math

import jax
import jax.numpy as jnp
from jax import lax
from jax.experimental import pallas as pl
from jax.experimental.pallas import tpu as pltpu
from jax.experimental.pallas import tpu_sc as plsc

D_MODEL = 1024
SCALE = math.sqrt(D_MODEL)
L = 16
NC, NS = 2, 16
NW = NC * NS
B_TOTAL = 4 * 4096
BPW = B_TOTAL // NW
C = 32
NCHUNK = BPW // C
NBUF = 3

_mesh = plsc.VectorSubcoreMesh(core_axis_name="c", subcore_axis_name="s")


@functools.partial(
    pl.kernel,
    mesh=_mesh,
    out_type=jax.ShapeDtypeStruct((B_TOTAL, D_MODEL), jnp.float32),
    scratch_types=[
        pltpu.VMEM((BPW,), jnp.int32),
        pltpu.VMEM((NBUF, C, D_MODEL), jnp.float32),
        pltpu.SemaphoreType.DMA,
        pltpu.SemaphoreType.DMA,
    ],
)
def _emb_lookup(table_hbm, idx_hbm, out_hbm, idx_v, rows_v, gsem, osem):
    wid = lax.axis_index("s") * NC + lax.axis_index("c")
    base = wid * BPW

    pltpu.sync_copy(idx_hbm.at[pl.ds(base, BPW)], idx_v)

    def gather_start(g):
        return pltpu.async_copy(
            table_hbm.at[idx_v.at[pl.ds(g * C, C)]], rows_v.at[g % NBUF], gsem
        )

    def scale_buf(buf):
        def row_body(r, _):
            for j in range(D_MODEL // L):
                sl = pl.ds(j * L, L)
                rows_v[buf, r, sl] = rows_v[buf, r, sl] * SCALE
            return 0

        lax.fori_loop(0, C, row_body, 0)

    gathers = [None] * NCHUNK
    out_copies = [None] * NBUF
    gathers[0] = gather_start(0)
    for g in range(NCHUNK):
        buf = g % NBUF
        gathers[g].wait()
        if g + 1 < NCHUNK:
            nbuf = (g + 1) % NBUF
            if out_copies[nbuf] is not None:
                out_copies[nbuf].wait()
                out_copies[nbuf] = None
            gathers[g + 1] = gather_start(g + 1)
        scale_buf(buf)
        out_copies[buf] = pltpu.async_copy(
            rows_v.at[buf], out_hbm.at[pl.ds(base + g * C, C)], osem
        )
    for b in range(NBUF):
        if out_copies[b] is not None:
            out_copies[b].wait()


def kernel(x, table):
    idx = x.astype(jnp.int32).reshape(B_TOTAL)
    out = _emb_lookup(table, idx)
    return out.reshape(x.shape + (D_MODEL,))
